# baseline (speedup 1.0000x reference)
import functools

import jax
import jax.numpy as jnp
from jax import lax
from jax.experimental import pallas as pl
from jax.experimental.pallas import tpu as pltpu
from jax.experimental.pallas import tpu_sc as plsc

N_S = 10000
N_T = 10000
D = 256
E1 = 160000
E2 = 160000

NC = 2
NST = 16
LANES = 16

NBC = 2
CBC = D // NBC

OFF_T = 10240
HALF = 10240
N1P = 20480
N2P = 10240
CH = 128
CCH = 80

TM = 512


def _sc_mesh():
  return plsc.VectorSubcoreMesh(
      core_axis_name="c", subcore_axis_name="s", num_cores=NC,
      num_subcores=NST)


def _fill_f32(ref, n, value):
  v = jnp.full((LANES,), value, jnp.float32)

  def body(i, carry):
    ref[pl.ds(i * LANES, LANES)] = v
    return carry

  lax.fori_loop(0, n // LANES, body, 0)


def _add_offset(src_ref, dst_ref, n, off):
  def body(i, carry):
    v = src_ref[pl.ds(i * LANES, LANES)]
    dst_ref[pl.ds(i * LANES, LANES)] = v + off
    return carry

  lax.fori_loop(0, n // LANES, body, 0)


def _deg_body(e0_hbm, e1_hbm, p1_hbm, deg1_hbm, deg2_hbm, e1a_hbm,
              hist, hist2, ia, ib, icadj, ones_buf, zbuf,
              idx_ta, idx_tb, idx_tc, ones_t,
              sa, sb, ss1, ss2):
  c = lax.axis_index("c")
  s = lax.axis_index("s")
  rows1 = HALF // NST

  _fill_f32(ones_buf, CH, 1.0)
  _fill_f32(ones_t, LANES, 1.0)
  _fill_f32(zbuf, rows1, 0.0)

  pltpu.sync_copy(zbuf.at[pl.ds(0, rows1)], hist.at[pl.ds(s * rows1, rows1)])
  pltpu.sync_copy(zbuf.at[pl.ds(0, rows1)], hist2.at[pl.ds(s * rows1, rows1)])
  plsc.subcore_barrier()

  epp = E1 // NST
  nchunks = epp // CH
  rem = epp - nchunks * CH
  ebase = s * epp

  def run(is_sc0):
    srcA = e0_hbm if is_sc0 else e1_hbm
    srcB = e1_hbm if is_sc0 else p1_hbm

    def start_idx(i, b):
      off = ebase + i * CH
      pltpu.async_copy(srcA.at[pl.ds(off, CH)], ia.at[b], sa.at[b])
      pltpu.async_copy(srcB.at[pl.ds(off, CH)], ib.at[b], sb.at[b])

    def wait_idx(b):
      pltpu.make_async_copy(srcA.at[pl.ds(0, CH)], ia.at[b], sa.at[b]).wait()
      pltpu.make_async_copy(srcB.at[pl.ds(0, CH)], ib.at[b], sb.at[b]).wait()

    def fire(i, b):
      pltpu.async_copy(ones_buf, hist.at[ia.at[b]], ss1.at[b], add=True)
      if is_sc0:
        _add_offset(ib.at[b], icadj.at[b], CH, OFF_T)
        pltpu.async_copy(icadj.at[b], e1a_hbm.at[pl.ds(ebase + i * CH, CH)],
                         ss2.at[b])
      else:
        pltpu.async_copy(ones_buf, hist2.at[ib.at[b]], ss2.at[b], add=True)

    def wait_fire(b):
      pltpu.make_async_copy(ones_buf, hist.at[ia.at[b]], ss1.at[b]).wait()
      if is_sc0:
        pltpu.make_async_copy(icadj.at[b], e1a_hbm.at[pl.ds(0, CH)],
                              ss2.at[b]).wait()
      else:
        pltpu.make_async_copy(ones_buf, hist2.at[ib.at[b]], ss2.at[b]).wait()

    def section(i, b):
      wait_idx(b)
      fire(i, b)

      @pl.when(i > 0)
      def _():
        wait_fire(1 - b)

      @pl.when(i + 1 < nchunks)
      def _():
        start_idx(i + 1, 1 - b)

    start_idx(0, 0)

    def pair(i2, carry):
      section(2 * i2, 0)
      section(2 * i2 + 1, 1)
      return carry

    lax.fori_loop(0, nchunks // 2, pair, 0)
    if nchunks % 2 == 1:
      section(nchunks - 1, 0)
      wait_fire(0)
    else:
      wait_fire(1)

    if rem:
      assert rem == LANES
      off = ebase + nchunks * CH
      pltpu.sync_copy(srcA.at[pl.ds(off, rem)], idx_ta)
      pltpu.sync_copy(srcB.at[pl.ds(off, rem)], idx_tb)
      pltpu.sync_copy(ones_t, hist.at[idx_ta], add=True)
      if is_sc0:
        _add_offset(idx_tb, idx_tc, rem, OFF_T)
        pltpu.sync_copy(idx_tc, e1a_hbm.at[pl.ds(off, rem)])
      else:
        pltpu.sync_copy(ones_t, hist2.at[idx_tb], add=True)

  @pl.when(c == 0)
  def _():
    run(True)

  @pl.when(c == 1)
  def _():
    run(False)

  plsc.subcore_barrier()

  @pl.when(c == 0)
  def _():
    pltpu.sync_copy(hist.at[pl.ds(s * rows1, rows1)],
                    deg1_hbm.at[pl.ds(s * rows1, rows1)])

  @pl.when(c == 1)
  def _():
    pltpu.sync_copy(hist.at[pl.ds(s * rows1, rows1)],
                    deg1_hbm.at[pl.ds(OFF_T + s * rows1, rows1)])
    pltpu.sync_copy(hist2.at[pl.ds(s * rows1, rows1)],
                    deg2_hbm.at[pl.ds(s * rows1, rows1)])


def _deg_call(e0, e1, p1):
  kfn = pl.kernel(
      _deg_body,
      out_type=(jax.ShapeDtypeStruct((N1P,), jnp.float32),
                jax.ShapeDtypeStruct((N2P,), jnp.float32),
                jax.ShapeDtypeStruct((E1,), jnp.int32)),
      mesh=_sc_mesh(),
      scratch_types=[
          pltpu.VMEM_SHARED((HALF,), jnp.float32),
          pltpu.VMEM_SHARED((HALF,), jnp.float32),
          pltpu.VMEM((2, CH), jnp.int32),
          pltpu.VMEM((2, CH), jnp.int32),
          pltpu.VMEM((2, CH), jnp.int32),
          pltpu.VMEM((CH,), jnp.float32),
          pltpu.VMEM((HALF // NST,), jnp.float32),
          pltpu.VMEM((LANES,), jnp.int32),
          pltpu.VMEM((LANES,), jnp.int32),
          pltpu.VMEM((LANES,), jnp.int32),
          pltpu.VMEM((LANES,), jnp.float32),
          pltpu.SemaphoreType.DMA((2,)),
          pltpu.SemaphoreType.DMA((2,)),
          pltpu.SemaphoreType.DMA((2,)),
          pltpu.SemaphoreType.DMA((2,)),
      ],
  )
  return kfn(e0, e1, p1)


def _conv_body(np_full, epp, passes,
               table_hbm, e0_hbm, e1_hbm, e1a_hbm, out_hbm,
               accum, gidx0, gidx1, sidx0, sidx1,
               rows0, rows1, idx_tg, idx_ts, rows_t,
               smgi0, smgi1, smsi0, smsi1, smg0, smg1, sms0, sms1):
  c = lax.axis_index("c")
  s = lax.axis_index("s")
  rslice = HALF // NST
  nsec = (epp // CCH) // 2
  nchunks = nsec * 2
  rem = epp - nchunks * CCH
  gidx = (gidx0, gidx1)
  sidx = (sidx0, sidx1)
  rows = (rows0, rows1)
  smgi = (smgi0, smgi1)
  smsi = (smsi0, smsi1)
  smg = (smg0, smg1)
  sms = (sms0, sms1)

  for g_arr, s_arr, base in passes:
    g_hbm = (e0_hbm, e1_hbm, e1a_hbm)[g_arr]
    s_hbm = (e0_hbm, e1_hbm, e1a_hbm)[s_arr]
    ebase = s * epp

    pltpu.sync_copy(table_hbm.at[c, pl.ds(base + s * rslice, rslice)],
                    accum.at[pl.ds(s * rslice, rslice)])
    plsc.subcore_barrier()

    def start_idx(i, b):
      for u in range(2):
        off = ebase + (2 * i + u) * CCH
        pltpu.async_copy(g_hbm.at[pl.ds(off, CCH)], gidx[u].at[b],
                         smgi[u].at[b])
        pltpu.async_copy(s_hbm.at[pl.ds(off, CCH)], sidx[u].at[b],
                         smsi[u].at[b])

    def wait_idx(b):
      for u in range(2):
        pltpu.make_async_copy(g_hbm.at[pl.ds(0, CCH)], gidx[u].at[b],
                              smgi[u].at[b]).wait()
        pltpu.make_async_copy(s_hbm.at[pl.ds(0, CCH)], sidx[u].at[b],
                              smsi[u].at[b]).wait()

    def wait_scatters(b):
      for u in range(2):
        pltpu.make_async_copy(rows[u].at[b], accum.at[sidx[u].at[b]],
                              sms[u].at[b]).wait()

    def section(i, b):
      wait_idx(b)
      for u in range(2):
        pltpu.async_copy(table_hbm.at[c].at[gidx[u].at[b]], rows[u].at[b],
                         smg[u].at[b])

      @pl.when(i > 0)
      def _():
        wait_scatters(1 - b)

      @pl.when(i + 1 < nsec)
      def _():
        start_idx(i + 1, 1 - b)

      for u in range(2):
        pltpu.make_async_copy(table_hbm.at[c].at[gidx[u].at[b]], rows[u].at[b],
                              smg[u].at[b]).wait()
        pltpu.async_copy(rows[u].at[b], accum.at[sidx[u].at[b]],
                         sms[u].at[b], add=True)

    start_idx(0, 0)

    def pair(i2, carry):
      section(2 * i2, 0)
      section(2 * i2 + 1, 1)
      return carry

    lax.fori_loop(0, nsec // 2, pair, 0)
    if nsec % 2 == 1:
      section(nsec - 1, 0)
      wait_scatters(0)
    else:
      wait_scatters(1)

    if rem:
      assert rem % LANES == 0
      for t in range(rem // LANES):
        off = ebase + nchunks * CCH + t * LANES
        pltpu.sync_copy(g_hbm.at[pl.ds(off, LANES)], idx_tg)
        pltpu.sync_copy(s_hbm.at[pl.ds(off, LANES)], idx_ts)
        pltpu.sync_copy(table_hbm.at[c].at[idx_tg], rows_t)
        pltpu.sync_copy(rows_t, accum.at[idx_ts], add=True)

    plsc.subcore_barrier()
    pltpu.sync_copy(accum.at[pl.ds(s * rslice, rslice)],
                    out_hbm.at[c, pl.ds(base + s * rslice, rslice)])
    plsc.subcore_barrier()


def _conv_call(table, e0, e1, e1a, np_full, passes):
  epp = E1 // NST
  body = functools.partial(_conv_body, np_full, epp, passes)
  kfn = pl.kernel(
      body,
      out_type=jax.ShapeDtypeStruct((NBC, np_full, CBC), jnp.float32),
      mesh=_sc_mesh(),
      scratch_types=[
          pltpu.VMEM_SHARED((HALF, CBC), jnp.float32),
          pltpu.VMEM((2, CCH), jnp.int32),
          pltpu.VMEM((2, CCH), jnp.int32),
          pltpu.VMEM((2, CCH), jnp.int32),
          pltpu.VMEM((2, CCH), jnp.int32),
          pltpu.VMEM((2, CCH, CBC), jnp.float32),
          pltpu.VMEM((2, CCH, CBC), jnp.float32),
          pltpu.VMEM((LANES,), jnp.int32),
          pltpu.VMEM((LANES,), jnp.int32),
          pltpu.VMEM((LANES, CBC), jnp.float32),
          pltpu.SemaphoreType.DMA((2,)),
          pltpu.SemaphoreType.DMA((2,)),
          pltpu.SemaphoreType.DMA((2,)),
          pltpu.SemaphoreType.DMA((2,)),
          pltpu.SemaphoreType.DMA((2,)),
          pltpu.SemaphoreType.DMA((2,)),
          pltpu.SemaphoreType.DMA((2,)),
          pltpu.SemaphoreType.DMA((2,)),
      ],
  )
  return kfn(table, e0, e1, e1a)


def _m1_body(x_ref, w1_ref, ws_ref, bs_ref, deg_ref, out_ref, wc_s, bc_s):
  cb = pl.program_id(0)
  m = pl.program_id(1)

  @pl.when(jnp.logical_and(cb == 0, m == 0))
  def _():
    w1 = w1_ref[...]
    wc_s[...] = jnp.dot(w1, ws_ref[...], preferred_element_type=jnp.float32)
    bc_s[...] = lax.dot_general(bs_ref[...], w1, (((1,), (1,)), ((), ())),
                                preferred_element_type=jnp.float32)

  dis = lax.rsqrt(deg_ref[...] + 1.0)
  wcblk = wc_s[pl.ds(cb * CBC, CBC), :]
  bcblk = bc_s[:, pl.ds(cb * CBC, CBC)]
  xw = lax.dot_general(x_ref[...], wcblk, (((1,), (1,)), ((), ())),
                       preferred_element_type=jnp.float32)
  out_ref[0] = (xw + bcblk) * dis


def _m1(x, W1, Ws, bs, deg1):
  grid = (NBC, N1P // TM)
  return pl.pallas_call(
      _m1_body,
      grid=grid,
      in_specs=[
          pl.BlockSpec((TM, D), lambda cb, m: (m, 0)),
          pl.BlockSpec((D, D), lambda cb, m: (0, 0)),
          pl.BlockSpec((D, D), lambda cb, m: (0, 0)),
          pl.BlockSpec((1, D), lambda cb, m: (0, 0)),
          pl.BlockSpec((TM, 1), lambda cb, m: (m, 0)),
      ],
      out_specs=pl.BlockSpec((1, TM, CBC), lambda cb, m: (cb, m, 0)),
      out_shape=jax.ShapeDtypeStruct((NBC, N1P, CBC), jnp.float32),
      scratch_shapes=[
          pltpu.VMEM((D, D), jnp.float32),
          pltpu.VMEM((1, D), jnp.float32),
      ],
  )(x, W1, Ws, bs.reshape(1, D), deg1)


def _m2b_body(acc_ref, deg1_ref, w2_ref, deg2_ref, acct_ref, deg1t_ref,
              out_ref, ht_ref):
  dis1 = lax.rsqrt(deg1_ref[...] + 1.0)
  dis2 = lax.rsqrt(deg2_ref[...] + 1.0)
  acc = jnp.zeros((TM, CBC), jnp.float32)
  for kb in range(NBC):
    h = jnp.maximum(acc_ref[kb] * dis1, 0.0)
    w2sub = w2_ref[:, kb * CBC:(kb + 1) * CBC]
    acc = acc + lax.dot_general(h, w2sub, (((1,), (1,)), ((), ())),
                                preferred_element_type=jnp.float32)
  out_ref[0] = acc * dis2
  dis1t = lax.rsqrt(deg1t_ref[...] + 1.0)
  ht_ref[...] = jnp.maximum(acct_ref[0] * dis1t, 0.0)


def _m2b(acc1, deg1, W2, deg2):
  grid = (NBC, N2P // TM)
  ro = OFF_T // TM
  return pl.pallas_call(
      _m2b_body,
      grid=grid,
      in_specs=[
          pl.BlockSpec((NBC, TM, CBC), lambda cb, m: (0, m, 0)),
          pl.BlockSpec((TM, 1), lambda cb, m: (m, 0)),
          pl.BlockSpec((CBC, D), lambda cb, m: (cb, 0)),
          pl.BlockSpec((TM, 1), lambda cb, m: (m, 0)),
          pl.BlockSpec((1, TM, CBC), lambda cb, m: (cb, ro + m, 0)),
          pl.BlockSpec((TM, 1), lambda cb, m: (ro + m, 0)),
      ],
      out_specs=(pl.BlockSpec((1, TM, CBC), lambda cb, m: (cb, m, 0)),
                 pl.BlockSpec((TM, CBC), lambda cb, m: (m, cb))),
      out_shape=(jax.ShapeDtypeStruct((NBC, N2P, CBC), jnp.float32),
                 jax.ShapeDtypeStruct((N2P, D), jnp.float32)),
  )(acc1, deg1, W2, deg2, acc1, deg1)


def _assemble_body(acc_ref, deg_ref, out_ref):
  dis = lax.rsqrt(deg_ref[...] + 1.0)
  for cb in range(NBC):
    out_ref[:, cb * CBC:(cb + 1) * CBC] = jnp.maximum(acc_ref[cb] * dis, 0.0)


def _assemble(acc, deg, row_off, nrows):
  grid = (nrows // TM,)
  ro = row_off // TM
  return pl.pallas_call(
      _assemble_body,
      grid=grid,
      in_specs=[
          pl.BlockSpec((NBC, TM, CBC), lambda m: (0, ro + m, 0)),
          pl.BlockSpec((TM, 1), lambda m: (ro + m, 0)),
      ],
      out_specs=pl.BlockSpec((TM, D), lambda m: (m, 0)),
      out_shape=jax.ShapeDtypeStruct((nrows, D), jnp.float32),
  )(acc, deg)


def kernel(edge_index, paper_edge_index, x_s, x_t, Ws, bs, W1, W2):
  e0 = edge_index[0].astype(jnp.int32)
  e1 = edge_index[1].astype(jnp.int32)
  p0 = paper_edge_index[0].astype(jnp.int32)
  p1 = paper_edge_index[1].astype(jnp.int32)

  x = jnp.zeros((N1P, D), jnp.float32)
  x = lax.dynamic_update_slice(x, x_s, (0, 0))
  x = lax.dynamic_update_slice(x, x_t, (OFF_T, 0))

  deg1, deg2, e1a = _deg_call(e0, e1, p1)
  deg1_2d = deg1.reshape(N1P, 1)
  deg2_2d = deg2.reshape(N2P, 1)
  xw1p = _m1(x, W1, Ws, bs, deg1_2d)
  acc1 = _conv_call(xw1p, e0, e1, e1a, N1P, ((2, 0, 0), (0, 1, OFF_T)))

  xw2p, ht = _m2b(acc1, deg1_2d, W2, deg2_2d)
  acc2 = _conv_call(xw2p, p0, p1, p0, N2P, ((0, 1, 0),))
  hs = _assemble(acc2, deg2_2d, 0, N2P)[:N_S]

  return (hs, ht[:N_T])


# speedup vs baseline: 1.0152x; 1.0152x over previous
import functools

import jax
import jax.numpy as jnp
from jax import lax
from jax.experimental import pallas as pl
from jax.experimental.pallas import tpu as pltpu
from jax.experimental.pallas import tpu_sc as plsc

N_S = 10000
N_T = 10000
D = 256
E1 = 160000
E2 = 160000

NC = 2
NST = 16
LANES = 16

NBC = 2
CBC = D // NBC

OFF_T = 10240
HALF = 10240
N1P = 20480
N2P = 10240
CH = 128
CCH = 80

TM = 512


def _sc_mesh():
  return plsc.VectorSubcoreMesh(
      core_axis_name="c", subcore_axis_name="s", num_cores=NC,
      num_subcores=NST)


def _fill_f32(ref, n, value):
  v = jnp.full((LANES,), value, jnp.float32)

  def body(i, carry):
    ref[pl.ds(i * LANES, LANES)] = v
    return carry

  lax.fori_loop(0, n // LANES, body, 0)


def _add_offset(src_ref, dst_ref, n, off):
  def body(i, carry):
    v = src_ref[pl.ds(i * LANES, LANES)]
    dst_ref[pl.ds(i * LANES, LANES)] = v + off
    return carry

  lax.fori_loop(0, n // LANES, body, 0)


def _deg_body(e0_hbm, e1_hbm, p1_hbm, deg1_hbm, deg2_hbm, e1a_hbm,
              hist, hist2, ia, ib, icadj, ones_buf, zbuf,
              idx_ta, idx_tb, idx_tc, ones_t,
              sa, sb, ss1, ss2):
  c = lax.axis_index("c")
  s = lax.axis_index("s")
  rows1 = HALF // NST

  _fill_f32(ones_buf, CH, 1.0)
  _fill_f32(ones_t, LANES, 1.0)
  _fill_f32(zbuf, rows1, 0.0)

  pltpu.sync_copy(zbuf.at[pl.ds(0, rows1)], hist.at[pl.ds(s * rows1, rows1)])
  pltpu.sync_copy(zbuf.at[pl.ds(0, rows1)], hist2.at[pl.ds(s * rows1, rows1)])
  plsc.subcore_barrier()

  epp = E1 // NST
  nchunks = epp // CH
  rem = epp - nchunks * CH
  ebase = s * epp

  def run(is_sc0):
    srcA = e0_hbm if is_sc0 else e1_hbm
    srcB = e1_hbm if is_sc0 else p1_hbm

    def start_idx(i, b):
      off = ebase + i * CH
      pltpu.async_copy(srcA.at[pl.ds(off, CH)], ia.at[b], sa.at[b])
      pltpu.async_copy(srcB.at[pl.ds(off, CH)], ib.at[b], sb.at[b])

    def wait_idx(b):
      pltpu.make_async_copy(srcA.at[pl.ds(0, CH)], ia.at[b], sa.at[b]).wait()
      pltpu.make_async_copy(srcB.at[pl.ds(0, CH)], ib.at[b], sb.at[b]).wait()

    def fire(i, b):
      pltpu.async_copy(ones_buf, hist.at[ia.at[b]], ss1.at[b], add=True)
      if is_sc0:
        _add_offset(ib.at[b], icadj.at[b], CH, OFF_T)
        pltpu.async_copy(icadj.at[b], e1a_hbm.at[pl.ds(ebase + i * CH, CH)],
                         ss2.at[b])
      else:
        pltpu.async_copy(ones_buf, hist2.at[ib.at[b]], ss2.at[b], add=True)

    def wait_fire(b):
      pltpu.make_async_copy(ones_buf, hist.at[ia.at[b]], ss1.at[b]).wait()
      if is_sc0:
        pltpu.make_async_copy(icadj.at[b], e1a_hbm.at[pl.ds(0, CH)],
                              ss2.at[b]).wait()
      else:
        pltpu.make_async_copy(ones_buf, hist2.at[ib.at[b]], ss2.at[b]).wait()

    def section(i, b):
      wait_idx(b)
      fire(i, b)

      @pl.when(i > 0)
      def _():
        wait_fire(1 - b)

      @pl.when(i + 1 < nchunks)
      def _():
        start_idx(i + 1, 1 - b)

    start_idx(0, 0)

    def pair(i2, carry):
      section(2 * i2, 0)
      section(2 * i2 + 1, 1)
      return carry

    lax.fori_loop(0, nchunks // 2, pair, 0)
    if nchunks % 2 == 1:
      section(nchunks - 1, 0)
      wait_fire(0)
    else:
      wait_fire(1)

    if rem:
      assert rem == LANES
      off = ebase + nchunks * CH
      pltpu.sync_copy(srcA.at[pl.ds(off, rem)], idx_ta)
      pltpu.sync_copy(srcB.at[pl.ds(off, rem)], idx_tb)
      pltpu.sync_copy(ones_t, hist.at[idx_ta], add=True)
      if is_sc0:
        _add_offset(idx_tb, idx_tc, rem, OFF_T)
        pltpu.sync_copy(idx_tc, e1a_hbm.at[pl.ds(off, rem)])
      else:
        pltpu.sync_copy(ones_t, hist2.at[idx_tb], add=True)

  @pl.when(c == 0)
  def _():
    run(True)

  @pl.when(c == 1)
  def _():
    run(False)

  plsc.subcore_barrier()

  @pl.when(c == 0)
  def _():
    pltpu.sync_copy(hist.at[pl.ds(s * rows1, rows1)],
                    deg1_hbm.at[pl.ds(s * rows1, rows1)])

  @pl.when(c == 1)
  def _():
    pltpu.sync_copy(hist.at[pl.ds(s * rows1, rows1)],
                    deg1_hbm.at[pl.ds(OFF_T + s * rows1, rows1)])
    pltpu.sync_copy(hist2.at[pl.ds(s * rows1, rows1)],
                    deg2_hbm.at[pl.ds(s * rows1, rows1)])


def _deg_call(e0, e1, p1):
  kfn = pl.kernel(
      _deg_body,
      out_type=(jax.ShapeDtypeStruct((N1P,), jnp.float32),
                jax.ShapeDtypeStruct((N2P,), jnp.float32),
                jax.ShapeDtypeStruct((E1,), jnp.int32)),
      mesh=_sc_mesh(),
      scratch_types=[
          pltpu.VMEM_SHARED((HALF,), jnp.float32),
          pltpu.VMEM_SHARED((HALF,), jnp.float32),
          pltpu.VMEM((2, CH), jnp.int32),
          pltpu.VMEM((2, CH), jnp.int32),
          pltpu.VMEM((2, CH), jnp.int32),
          pltpu.VMEM((CH,), jnp.float32),
          pltpu.VMEM((HALF // NST,), jnp.float32),
          pltpu.VMEM((LANES,), jnp.int32),
          pltpu.VMEM((LANES,), jnp.int32),
          pltpu.VMEM((LANES,), jnp.int32),
          pltpu.VMEM((LANES,), jnp.float32),
          pltpu.SemaphoreType.DMA((2,)),
          pltpu.SemaphoreType.DMA((2,)),
          pltpu.SemaphoreType.DMA((2,)),
          pltpu.SemaphoreType.DMA((2,)),
      ],
  )
  return kfn(e0, e1, p1)


def _conv_body(np_full, epp, passes,
               table_hbm, e0_hbm, e1_hbm, e1a_hbm, out_hbm,
               accum, gidx0, gidx1, sidx0, sidx1,
               rows0, rows1, idx_tg, idx_ts, rows_t,
               smgi0, smgi1, smsi0, smsi1, smg0, smg1, sms0, sms1):
  c = lax.axis_index("c")
  s = lax.axis_index("s")
  rslice = HALF // NST
  nsec = (epp // CCH) // 2
  nchunks = nsec * 2
  rem = epp - nchunks * CCH
  gidx = (gidx0, gidx1)
  sidx = (sidx0, sidx1)
  rows = (rows0, rows1)
  smgi = (smgi0, smgi1)
  smsi = (smsi0, smsi1)
  smg = (smg0, smg1)
  sms = (sms0, sms1)

  for g_arr, s_arr, base in passes:
    g_hbm = (e0_hbm, e1_hbm, e1a_hbm)[g_arr]
    s_hbm = (e0_hbm, e1_hbm, e1a_hbm)[s_arr]
    ebase = s * epp

    pltpu.sync_copy(table_hbm.at[c, pl.ds(base + s * rslice, rslice)],
                    accum.at[pl.ds(s * rslice, rslice)])
    plsc.subcore_barrier()

    def start_idx(i, b):
      for u in range(2):
        off = ebase + (2 * i + u) * CCH
        pltpu.async_copy(g_hbm.at[pl.ds(off, CCH)], gidx[u].at[b],
                         smgi[u].at[b])
        pltpu.async_copy(s_hbm.at[pl.ds(off, CCH)], sidx[u].at[b],
                         smsi[u].at[b])

    def wait_idx(b):
      for u in range(2):
        pltpu.make_async_copy(g_hbm.at[pl.ds(0, CCH)], gidx[u].at[b],
                              smgi[u].at[b]).wait()
        pltpu.make_async_copy(s_hbm.at[pl.ds(0, CCH)], sidx[u].at[b],
                              smsi[u].at[b]).wait()

    def wait_scatters(b):
      for u in range(2):
        pltpu.make_async_copy(rows[u].at[b], accum.at[sidx[u].at[b]],
                              sms[u].at[b]).wait()

    def section(i, b):
      wait_idx(b)
      for u in range(2):
        pltpu.async_copy(table_hbm.at[c].at[gidx[u].at[b]], rows[u].at[b],
                         smg[u].at[b])

      @pl.when(i > 0)
      def _():
        wait_scatters(1 - b)

      @pl.when(i + 1 < nsec)
      def _():
        start_idx(i + 1, 1 - b)

      for u in range(2):
        pltpu.make_async_copy(table_hbm.at[c].at[gidx[u].at[b]], rows[u].at[b],
                              smg[u].at[b]).wait()
        pltpu.async_copy(rows[u].at[b], accum.at[sidx[u].at[b]],
                         sms[u].at[b], add=True)

    start_idx(0, 0)

    def pair(i2, carry):
      section(2 * i2, 0)
      section(2 * i2 + 1, 1)
      return carry

    lax.fori_loop(0, nsec // 2, pair, 0)
    if nsec % 2 == 1:
      section(nsec - 1, 0)
      wait_scatters(0)
    else:
      wait_scatters(1)

    if rem:
      assert rem % LANES == 0
      for t in range(rem // LANES):
        off = ebase + nchunks * CCH + t * LANES
        pltpu.sync_copy(g_hbm.at[pl.ds(off, LANES)], idx_tg)
        pltpu.sync_copy(s_hbm.at[pl.ds(off, LANES)], idx_ts)
        pltpu.sync_copy(table_hbm.at[c].at[idx_tg], rows_t)
        pltpu.sync_copy(rows_t, accum.at[idx_ts], add=True)

    plsc.subcore_barrier()
    pltpu.sync_copy(accum.at[pl.ds(s * rslice, rslice)],
                    out_hbm.at[c, pl.ds(base + s * rslice, rslice)])
    plsc.subcore_barrier()


def _conv_call(table, e0, e1, e1a, np_full, passes):
  epp = E1 // NST
  body = functools.partial(_conv_body, np_full, epp, passes)
  kfn = pl.kernel(
      body,
      out_type=jax.ShapeDtypeStruct((NBC, np_full, CBC), jnp.float32),
      mesh=_sc_mesh(),
      scratch_types=[
          pltpu.VMEM_SHARED((HALF, CBC), jnp.float32),
          pltpu.VMEM((2, CCH), jnp.int32),
          pltpu.VMEM((2, CCH), jnp.int32),
          pltpu.VMEM((2, CCH), jnp.int32),
          pltpu.VMEM((2, CCH), jnp.int32),
          pltpu.VMEM((2, CCH, CBC), jnp.float32),
          pltpu.VMEM((2, CCH, CBC), jnp.float32),
          pltpu.VMEM((LANES,), jnp.int32),
          pltpu.VMEM((LANES,), jnp.int32),
          pltpu.VMEM((LANES, CBC), jnp.float32),
          pltpu.SemaphoreType.DMA((2,)),
          pltpu.SemaphoreType.DMA((2,)),
          pltpu.SemaphoreType.DMA((2,)),
          pltpu.SemaphoreType.DMA((2,)),
          pltpu.SemaphoreType.DMA((2,)),
          pltpu.SemaphoreType.DMA((2,)),
          pltpu.SemaphoreType.DMA((2,)),
          pltpu.SemaphoreType.DMA((2,)),
      ],
  )
  return kfn(table, e0, e1, e1a)


def _wfuse_body(w1_ref, ws_ref, bs_ref, wc_ref, bc_ref):
  w1 = w1_ref[...]
  wc_ref[...] = jnp.dot(w1, ws_ref[...], preferred_element_type=jnp.float32)
  bc_ref[...] = lax.dot_general(bs_ref[...], w1, (((1,), (1,)), ((), ())),
                                preferred_element_type=jnp.float32)


def _wfuse(W1, Ws, bs):
  return pl.pallas_call(
      _wfuse_body,
      out_shape=(jax.ShapeDtypeStruct((D, D), jnp.float32),
                 jax.ShapeDtypeStruct((1, D), jnp.float32)),
  )(W1, Ws, bs.reshape(1, D))


def _m1_body(x_ref, wc_ref, bc_ref, deg_ref, out_ref):
  dis = lax.rsqrt(deg_ref[...] + 1.0)
  xw = lax.dot_general(x_ref[...], wc_ref[...], (((1,), (1,)), ((), ())),
                       preferred_element_type=jnp.float32)
  out_ref[0] = (xw + bc_ref[0]) * dis


def _m1(x, wc, bc, deg1):
  grid = (NBC, N1P // TM)
  return pl.pallas_call(
      _m1_body,
      grid=grid,
      in_specs=[
          pl.BlockSpec((TM, D), lambda cb, m: (m, 0)),
          pl.BlockSpec((CBC, D), lambda cb, m: (cb, 0)),
          pl.BlockSpec((1, 1, CBC), lambda cb, m: (cb, 0, 0)),
          pl.BlockSpec((TM, 1), lambda cb, m: (m, 0)),
      ],
      out_specs=pl.BlockSpec((1, TM, CBC), lambda cb, m: (cb, m, 0)),
      out_shape=jax.ShapeDtypeStruct((NBC, N1P, CBC), jnp.float32),
  )(x, wc, bc.reshape(NBC, 1, CBC), deg1)


def _m2b_body(acc_ref, deg1_ref, w2_ref, deg2_ref, acct_ref, deg1t_ref,
              out_ref, ht_ref):
  dis1 = lax.rsqrt(deg1_ref[...] + 1.0)
  dis2 = lax.rsqrt(deg2_ref[...] + 1.0)
  acc = jnp.zeros((TM, CBC), jnp.float32)
  for kb in range(NBC):
    h = jnp.maximum(acc_ref[kb] * dis1, 0.0)
    w2sub = w2_ref[:, kb * CBC:(kb + 1) * CBC]
    acc = acc + lax.dot_general(h, w2sub, (((1,), (1,)), ((), ())),
                                preferred_element_type=jnp.float32)
  out_ref[0] = acc * dis2
  dis1t = lax.rsqrt(deg1t_ref[...] + 1.0)
  ht_ref[...] = jnp.maximum(acct_ref[0] * dis1t, 0.0)


def _m2b(acc1, deg1, W2, deg2):
  grid = (NBC, N2P // TM)
  ro = OFF_T // TM
  return pl.pallas_call(
      _m2b_body,
      grid=grid,
      in_specs=[
          pl.BlockSpec((NBC, TM, CBC), lambda cb, m: (0, m, 0)),
          pl.BlockSpec((TM, 1), lambda cb, m: (m, 0)),
          pl.BlockSpec((CBC, D), lambda cb, m: (cb, 0)),
          pl.BlockSpec((TM, 1), lambda cb, m: (m, 0)),
          pl.BlockSpec((1, TM, CBC), lambda cb, m: (cb, ro + m, 0)),
          pl.BlockSpec((TM, 1), lambda cb, m: (ro + m, 0)),
      ],
      out_specs=(pl.BlockSpec((1, TM, CBC), lambda cb, m: (cb, m, 0)),
                 pl.BlockSpec((TM, CBC), lambda cb, m: (m, cb))),
      out_shape=(jax.ShapeDtypeStruct((NBC, N2P, CBC), jnp.float32),
                 jax.ShapeDtypeStruct((N_T, D), jnp.float32)),
  )(acc1, deg1, W2, deg2, acc1, deg1)


def _assemble_body(acc_ref, deg_ref, out_ref):
  dis = lax.rsqrt(deg_ref[...] + 1.0)
  for cb in range(NBC):
    out_ref[:, cb * CBC:(cb + 1) * CBC] = jnp.maximum(acc_ref[cb] * dis, 0.0)


TM2 = 400


def _assemble(acc, deg):
  grid = (N_S // TM2,)
  return pl.pallas_call(
      _assemble_body,
      grid=grid,
      in_specs=[
          pl.BlockSpec((NBC, TM2, CBC), lambda m: (0, m, 0)),
          pl.BlockSpec((TM2, 1), lambda m: (m, 0)),
      ],
      out_specs=pl.BlockSpec((TM2, D), lambda m: (m, 0)),
      out_shape=jax.ShapeDtypeStruct((N_S, D), jnp.float32),
  )(acc, deg)


def kernel(edge_index, paper_edge_index, x_s, x_t, Ws, bs, W1, W2):
  e0 = edge_index[0].astype(jnp.int32)
  e1 = edge_index[1].astype(jnp.int32)
  p0 = paper_edge_index[0].astype(jnp.int32)
  p1 = paper_edge_index[1].astype(jnp.int32)

  x = jnp.zeros((N1P, D), jnp.float32)
  x = lax.dynamic_update_slice(x, x_s, (0, 0))
  x = lax.dynamic_update_slice(x, x_t, (OFF_T, 0))

  deg1, deg2, e1a = _deg_call(e0, e1, p1)
  wc, bc = _wfuse(W1, Ws, bs)
  deg1_2d = deg1.reshape(N1P, 1)
  deg2_2d = deg2.reshape(N2P, 1)
  xw1p = _m1(x, wc, bc, deg1_2d)
  acc1 = _conv_call(xw1p, e0, e1, e1a, N1P, ((2, 0, 0), (0, 1, OFF_T)))

  xw2p, ht = _m2b(acc1, deg1_2d, W2, deg2_2d)
  acc2 = _conv_call(xw2p, p0, p1, p0, N2P, ((0, 1, 0),))
  hs = _assemble(acc2, deg2_2d)

  return (hs, ht)
